# Initial kernel scaffold; baseline (speedup 1.0000x reference)
import jax
import jax.numpy as jnp
from jax.experimental import pallas as pl


def kernel(features, edge_index, W1, b1, W2, b2):
    raise NotImplementedError("write your pallas kernel here")


# speedup vs baseline: 2.5473x; 2.5473x over previous
import functools

import jax
import jax.numpy as jnp
from jax import lax
from jax.experimental import pallas as pl
from jax.experimental.pallas import tpu as pltpu
from jax.experimental.pallas import tpu_sc as plsc

N = 10000
E = 160000
D_IN = 256
D_H = 256
D_OUT = 128

N_PAD = 10240
E_PAD = 163840
K = 128
NT = 16
NC = 2
ROWS_PER_TILE = N_PAD // NT
RBLK = 256
GRID = N_PAD // RBLK

_mesh = plsc.VectorSubcoreMesh(
    core_axis_name="c", subcore_axis_name="s", num_cores=NC, num_subcores=NT)


@functools.partial(
    pl.kernel,
    out_type=jax.ShapeDtypeStruct((NC * N_PAD,), jnp.float32),
    mesh=_mesh,
    scratch_types=[
        pltpu.VMEM_SHARED((N_PAD,), jnp.float32),
        pltpu.VMEM((K,), jnp.float32),
        pltpu.VMEM((K,), jnp.int32),
    ],
)
def _sc_degrees(src_hbm, dst_hbm, ones_hbm, zeros_hbm,
                out_ref, acc, onesb, idxb):
    c = lax.axis_index("c")
    s = lax.axis_index("s")
    sl = pl.ds(pl.multiple_of(s * ROWS_PER_TILE, 8), ROWS_PER_TILE)
    pltpu.sync_copy(zeros_hbm.at[sl], acc.at[sl])
    pltpu.sync_copy(ones_hbm, onesb)
    plsc.subcore_barrier()

    per_tile = E_PAD // NT
    base = s * per_tile

    def body(j, carry):
        st = pl.multiple_of(base + j * K, 8)
        @pl.when(c == 0)
        def _():
            pltpu.sync_copy(src_hbm.at[pl.ds(st, K)], idxb)

        @pl.when(c == 1)
        def _():
            pltpu.sync_copy(dst_hbm.at[pl.ds(st, K)], idxb)
        pltpu.sync_copy(onesb, acc.at[idxb], add=True)
        return carry

    lax.fori_loop(0, per_tile // K, body, 0)
    plsc.subcore_barrier()
    ob = pl.ds(pl.multiple_of(c * N_PAD + s * ROWS_PER_TILE, 8), ROWS_PER_TILE)
    pltpu.sync_copy(acc.at[sl], out_ref.at[ob])


def _make_edge_agg(table_rows, feature_split):
    per_tile = E_PAD // NT if feature_split else E_PAD // (NC * NT)
    nbatch = per_tile // K

    @functools.partial(
        pl.kernel,
        out_type=jax.ShapeDtypeStruct((NC * N_PAD, D_OUT), jnp.float32),
        mesh=_mesh,
        scratch_types=[
            pltpu.VMEM_SHARED((N_PAD, D_OUT), jnp.float32),
            pltpu.VMEM((K,), jnp.int32),
            pltpu.VMEM((K,), jnp.int32),
            pltpu.VMEM((K, D_OUT), jnp.float32),
            pltpu.SemaphoreType.DMA,
        ],
    )
    def agg(table_hbm, src_hbm, dst_hbm, zeros_hbm,
            out_hbm, acc, srcb, dstb, rows, sem):
        c = lax.axis_index("c")
        s = lax.axis_index("s")
        sl = pl.ds(pl.multiple_of(s * ROWS_PER_TILE, 8), ROWS_PER_TILE)
        pltpu.sync_copy(zeros_hbm.at[sl], acc.at[sl])
        plsc.subcore_barrier()

        if feature_split:
            base = s * per_tile
            off = c * N_PAD
        else:
            base = (c * NT + s) * per_tile
            off = None

        def body(j, carry):
            st = pl.multiple_of(base + j * K, 8)
            pltpu.sync_copy(src_hbm.at[pl.ds(st, K)], srcb)
            if off is not None:
                for k2 in range(K // 16):
                    chunk = pl.ds(k2 * 16, 16)
                    srcb[chunk] = srcb[chunk] + off
            pltpu.sync_copy(dst_hbm.at[pl.ds(st, K)], dstb)
            pltpu.async_copy(table_hbm.at[srcb], rows, sem).wait()
            pltpu.sync_copy(rows, acc.at[dstb], add=True)
            return carry

        lax.fori_loop(0, nbatch, body, 0)
        plsc.subcore_barrier()
        ob = pl.ds(pl.multiple_of(c * N_PAD + s * ROWS_PER_TILE, 8),
                   ROWS_PER_TILE)
        pltpu.sync_copy(acc.at[sl], out_hbm.at[ob])

    return agg


_edge_agg_l1 = _make_edge_agg(NC * N_PAD, feature_split=True)
_edge_agg_l2 = _make_edge_agg(N_PAD, feature_split=False)


def _elu(x):
    return jnp.where(x > 0, x, jnp.exp(jnp.minimum(x, 0.0)) - 1.0)


def _tc1_body(x_ref, w_ref, ps_ref, pd_ref, h1_ref, ns_ref, nd_ref):
    ns = lax.rsqrt(jnp.maximum(ps_ref[...], 1.0))
    nd = lax.rsqrt(jnp.maximum(pd_ref[...], 1.0))
    ns_ref[...] = ns
    nd_ref[...] = nd
    m = jnp.dot(x_ref[...], w_ref[...], preferred_element_type=jnp.float32)
    m = m * ns
    h1_ref[0] = m[:, :D_OUT]
    h1_ref[1] = m[:, D_OUT:]


def _tc1(x_pad, w1, ps, pd):
    return pl.pallas_call(
        _tc1_body,
        grid=(GRID,),
        in_specs=[
            pl.BlockSpec((RBLK, D_IN), lambda i: (i, 0)),
            pl.BlockSpec((D_IN, D_H), lambda i: (0, 0)),
            pl.BlockSpec((RBLK, 1), lambda i: (i, 0)),
            pl.BlockSpec((RBLK, 1), lambda i: (i, 0)),
        ],
        out_specs=[
            pl.BlockSpec((NC, RBLK, D_OUT), lambda i: (0, i, 0)),
            pl.BlockSpec((RBLK, 1), lambda i: (i, 0)),
            pl.BlockSpec((RBLK, 1), lambda i: (i, 0)),
        ],
        out_shape=[
            jax.ShapeDtypeStruct((NC, N_PAD, D_OUT), jnp.float32),
            jax.ShapeDtypeStruct((N_PAD, 1), jnp.float32),
            jax.ShapeDtypeStruct((N_PAD, 1), jnp.float32),
        ],
    )(x_pad, w1, ps, pd)


def _tc2_body(agg_ref, ns_ref, nd_ref, b1_ref, w2_ref, h2_ref):
    nd = nd_ref[...]
    ns = ns_ref[...]
    ta = _elu(agg_ref[0] * nd + b1_ref[0:1, 0, :]) * ns
    tb = _elu(agg_ref[1] * nd + b1_ref[0:1, 1, :]) * ns
    h2 = jnp.dot(ta, w2_ref[0], preferred_element_type=jnp.float32)
    h2 = h2 + jnp.dot(tb, w2_ref[1], preferred_element_type=jnp.float32)
    h2_ref[...] = h2


def _tc2(agg1, ns, nd, b1r, w2r):
    return pl.pallas_call(
        _tc2_body,
        grid=(GRID,),
        in_specs=[
            pl.BlockSpec((NC, RBLK, D_OUT), lambda i: (0, i, 0)),
            pl.BlockSpec((RBLK, 1), lambda i: (i, 0)),
            pl.BlockSpec((RBLK, 1), lambda i: (i, 0)),
            pl.BlockSpec((1, NC, D_OUT), lambda i: (0, 0, 0)),
            pl.BlockSpec((NC, D_OUT, D_OUT), lambda i: (0, 0, 0)),
        ],
        out_specs=pl.BlockSpec((RBLK, D_OUT), lambda i: (i, 0)),
        out_shape=jax.ShapeDtypeStruct((N_PAD, D_OUT), jnp.float32),
    )(agg1, ns, nd, b1r, w2r)


def _tc3_body(p_ref, nd_ref, b2_ref, out_ref):
    a = (p_ref[0] + p_ref[1]) * nd_ref[...] + b2_ref[...]
    out_ref[...] = _elu(a)


def _tc3(agg2, nd, b2r):
    return pl.pallas_call(
        _tc3_body,
        grid=(GRID,),
        in_specs=[
            pl.BlockSpec((NC, RBLK, D_OUT), lambda i: (0, i, 0)),
            pl.BlockSpec((RBLK, 1), lambda i: (i, 0)),
            pl.BlockSpec((1, D_OUT), lambda i: (0, 0)),
        ],
        out_specs=pl.BlockSpec((RBLK, D_OUT), lambda i: (i, 0)),
        out_shape=jax.ShapeDtypeStruct((N_PAD, D_OUT), jnp.float32),
    )(agg2, nd, b2r)


def kernel(features, edge_index, W1, b1, W2, b2):
    src = edge_index[0].astype(jnp.int32)
    dst = edge_index[1].astype(jnp.int32)
    pad = jnp.full((E_PAD - E,), N, dtype=jnp.int32)
    src_p = jnp.concatenate([src, pad])
    dst_p = jnp.concatenate([dst, pad])

    x_pad = jnp.pad(features, ((0, N_PAD - N), (0, 0)))
    ones1 = jnp.ones((K,), jnp.float32)
    zeros1 = jnp.zeros((N_PAD,), jnp.float32)
    zeros_t = jnp.zeros((N_PAD, D_OUT), jnp.float32)

    deg_f = _sc_degrees(src_p, dst_p, ones1, zeros1)
    ps = deg_f[:N_PAD].reshape(N_PAD, 1)
    pd = deg_f[N_PAD:].reshape(N_PAD, 1)

    h1, ns, nd = _tc1(x_pad, W1, ps, pd)

    agg1_f = _edge_agg_l1(h1.reshape(NC * N_PAD, D_OUT), src_p, dst_p, zeros_t)
    agg1 = agg1_f.reshape(NC, N_PAD, D_OUT)

    h2 = _tc2(agg1, ns, nd, b1.reshape(1, NC, D_OUT),
              W2.reshape(NC, D_OUT, D_OUT))

    agg2_f = _edge_agg_l2(h2, src_p, dst_p, zeros_t)
    agg2 = agg2_f.reshape(NC, N_PAD, D_OUT)

    logits = _tc3(agg2, nd, b2.reshape(1, D_OUT))
    return logits[:N]


# speedup vs baseline: 3.6451x; 1.4310x over previous
import functools

import jax
import jax.numpy as jnp
from jax import lax
from jax.experimental import pallas as pl
from jax.experimental.pallas import tpu as pltpu
from jax.experimental.pallas import tpu_sc as plsc

N = 10000
E = 160000
D_IN = 256
D_H = 256
D_OUT = 128

N_PAD = 10240
SEG = E // 32
SEG_PAD = 5120
E_PAD = 32 * SEG_PAD
K = 128
NT = 16
NC = 2
ROWS_PER_TILE = N_PAD // NT
RBLK = 256
GRID = N_PAD // RBLK

_mesh = plsc.VectorSubcoreMesh(
    core_axis_name="c", subcore_axis_name="s", num_cores=NC, num_subcores=NT)


@functools.partial(
    pl.kernel,
    out_type=jax.ShapeDtypeStruct((NC * N_PAD,), jnp.float32),
    mesh=_mesh,
    scratch_types=[
        pltpu.VMEM_SHARED((N_PAD,), jnp.float32),
        pltpu.VMEM((K,), jnp.float32),
        pltpu.VMEM((K,), jnp.int32),
    ],
)
def _sc_degrees(ep_hbm, ones_hbm, zeros_hbm, out_ref, acc, onesb, idxb):
    c = lax.axis_index("c")
    s = lax.axis_index("s")
    sl = pl.ds(pl.multiple_of(s * ROWS_PER_TILE, 8), ROWS_PER_TILE)
    pltpu.sync_copy(zeros_hbm.at[sl], acc.at[sl])
    pltpu.sync_copy(ones_hbm, onesb)
    plsc.subcore_barrier()

    per_tile = E_PAD // NT
    base = s * per_tile

    def body(j, carry):
        st = pl.multiple_of(base + j * K, 8)

        @pl.when(c == 0)
        def _():
            pltpu.sync_copy(ep_hbm.at[0, pl.ds(st, K)], idxb)

        @pl.when(c == 1)
        def _():
            pltpu.sync_copy(ep_hbm.at[1, pl.ds(st, K)], idxb)

        pltpu.sync_copy(onesb, acc.at[idxb], add=True)
        return carry

    lax.fori_loop(0, per_tile // K, body, 0)
    plsc.subcore_barrier()
    ob = pl.ds(pl.multiple_of(c * N_PAD + s * ROWS_PER_TILE, 8), ROWS_PER_TILE)
    pltpu.sync_copy(acc.at[sl], out_ref.at[ob])


def _make_edge_agg(feature_split):
    per_tile = E_PAD // NT if feature_split else E_PAD // (NC * NT)
    nbatch = per_tile // K

    @functools.partial(
        pl.kernel,
        out_type=jax.ShapeDtypeStruct((NC * N_PAD, D_OUT), jnp.float32),
        mesh=_mesh,
        scratch_types=[
            pltpu.VMEM_SHARED((N_PAD, D_OUT), jnp.float32),
            pltpu.VMEM((2, K), jnp.int32),
            pltpu.VMEM((2, K), jnp.int32),
            pltpu.VMEM((K, D_OUT), jnp.float32),
            pltpu.VMEM((K, D_OUT), jnp.float32),
            pltpu.SemaphoreType.DMA,
            pltpu.SemaphoreType.DMA,
        ],
    )
    def agg(table_hbm, ep_hbm, zeros_hbm,
            out_hbm, acc, idx_a, idx_b, rows_a, rows_b, sem_a, sem_b):
        c = lax.axis_index("c")
        s = lax.axis_index("s")
        sl = pl.ds(pl.multiple_of(s * ROWS_PER_TILE, 8), ROWS_PER_TILE)
        pltpu.sync_copy(zeros_hbm.at[sl], acc.at[sl])
        plsc.subcore_barrier()

        if feature_split:
            base = s * per_tile
        else:
            base = (c * NT + s) * per_tile
        off = c * N_PAD if feature_split else None
        last = E_PAD - K

        def load_idx(j, idxb):
            st = pl.multiple_of(jnp.minimum(base + j * K, last), 8)
            pltpu.sync_copy(ep_hbm.at[:, pl.ds(st, K)], idxb)
            if off is not None:
                for k2 in range(K // 16):
                    ch = pl.ds(k2 * 16, 16)
                    idxb[0, ch] = idxb[0, ch] + off

        def gather(idxb, rows, sem):
            pltpu.async_copy(table_hbm.at[idxb.at[0]], rows, sem)

        def gather_wait(idxb, rows, sem):
            pltpu.make_async_copy(table_hbm.at[idxb.at[0]], rows, sem).wait()

        def scatter(idxb, rows):
            pltpu.sync_copy(rows, acc.at[idxb.at[1]], add=True)

        load_idx(0, idx_a)
        gather(idx_a, rows_a, sem_a)
        load_idx(1, idx_b)

        def pair(j2, carry):
            j = j2 * 2
            gather_wait(idx_a, rows_a, sem_a)
            gather(idx_b, rows_b, sem_b)
            scatter(idx_a, rows_a)
            load_idx(j + 2, idx_a)
            gather_wait(idx_b, rows_b, sem_b)

            @pl.when(j + 2 < nbatch)
            def _():
                gather(idx_a, rows_a, sem_a)

            scatter(idx_b, rows_b)
            load_idx(j + 3, idx_b)
            return carry

        lax.fori_loop(0, nbatch // 2, pair, 0)
        plsc.subcore_barrier()
        ob = pl.ds(pl.multiple_of(c * N_PAD + s * ROWS_PER_TILE, 8),
                   ROWS_PER_TILE)
        pltpu.sync_copy(acc.at[sl], out_hbm.at[ob])

    return agg


_edge_agg_l1 = _make_edge_agg(feature_split=True)
_edge_agg_l2 = _make_edge_agg(feature_split=False)


def _elu(x):
    return jnp.where(x > 0, x, jnp.exp(jnp.minimum(x, 0.0)) - 1.0)


def _tc1_body(x_ref, w_ref, ps_ref, pd_ref, h1_ref, ns_ref, nd_ref):
    ns = lax.rsqrt(jnp.maximum(ps_ref[...], 1.0))
    nd = lax.rsqrt(jnp.maximum(pd_ref[...], 1.0))
    ns_ref[...] = ns
    nd_ref[...] = nd
    m = jnp.dot(x_ref[...], w_ref[...], preferred_element_type=jnp.float32)
    m = m * ns
    h1_ref[0] = m[:, :D_OUT]
    h1_ref[1] = m[:, D_OUT:]


def _tc1(x_pad, w1, ps, pd):
    return pl.pallas_call(
        _tc1_body,
        grid=(GRID,),
        in_specs=[
            pl.BlockSpec((RBLK, D_IN), lambda i: (i, 0)),
            pl.BlockSpec((D_IN, D_H), lambda i: (0, 0)),
            pl.BlockSpec((RBLK, 1), lambda i: (i, 0)),
            pl.BlockSpec((RBLK, 1), lambda i: (i, 0)),
        ],
        out_specs=[
            pl.BlockSpec((NC, RBLK, D_OUT), lambda i: (0, i, 0)),
            pl.BlockSpec((RBLK, 1), lambda i: (i, 0)),
            pl.BlockSpec((RBLK, 1), lambda i: (i, 0)),
        ],
        out_shape=[
            jax.ShapeDtypeStruct((NC, N_PAD, D_OUT), jnp.float32),
            jax.ShapeDtypeStruct((N_PAD, 1), jnp.float32),
            jax.ShapeDtypeStruct((N_PAD, 1), jnp.float32),
        ],
    )(x_pad, w1, ps, pd)


def _tc2_body(agg_ref, ns_ref, nd_ref, b1_ref, w2_ref, h2_ref):
    nd = nd_ref[...]
    ns = ns_ref[...]
    ta = _elu(agg_ref[0] * nd + b1_ref[0:1, 0, :]) * ns
    tb = _elu(agg_ref[1] * nd + b1_ref[0:1, 1, :]) * ns
    h2 = jnp.dot(ta, w2_ref[0], preferred_element_type=jnp.float32)
    h2 = h2 + jnp.dot(tb, w2_ref[1], preferred_element_type=jnp.float32)
    h2_ref[...] = h2


def _tc2(agg1, ns, nd, b1r, w2r):
    return pl.pallas_call(
        _tc2_body,
        grid=(GRID,),
        in_specs=[
            pl.BlockSpec((NC, RBLK, D_OUT), lambda i: (0, i, 0)),
            pl.BlockSpec((RBLK, 1), lambda i: (i, 0)),
            pl.BlockSpec((RBLK, 1), lambda i: (i, 0)),
            pl.BlockSpec((1, NC, D_OUT), lambda i: (0, 0, 0)),
            pl.BlockSpec((NC, D_OUT, D_OUT), lambda i: (0, 0, 0)),
        ],
        out_specs=pl.BlockSpec((RBLK, D_OUT), lambda i: (i, 0)),
        out_shape=jax.ShapeDtypeStruct((N_PAD, D_OUT), jnp.float32),
    )(agg1, ns, nd, b1r, w2r)


def _tc3_body(p_ref, nd_ref, b2_ref, out_ref):
    a = (p_ref[0] + p_ref[1]) * nd_ref[...] + b2_ref[...]
    out_ref[...] = _elu(a)


def _tc3(agg2, nd, b2r):
    return pl.pallas_call(
        _tc3_body,
        grid=(GRID,),
        in_specs=[
            pl.BlockSpec((NC, RBLK, D_OUT), lambda i: (0, i, 0)),
            pl.BlockSpec((RBLK, 1), lambda i: (i, 0)),
            pl.BlockSpec((1, D_OUT), lambda i: (0, 0)),
        ],
        out_specs=pl.BlockSpec((RBLK, D_OUT), lambda i: (i, 0)),
        out_shape=jax.ShapeDtypeStruct((N_PAD, D_OUT), jnp.float32),
    )(agg2, nd, b2r)


def kernel(features, edge_index, W1, b1, W2, b2):
    ep = edge_index.astype(jnp.int32).reshape(2, 32, SEG)
    pad_src = jnp.full((SEG_PAD - SEG,), N, dtype=jnp.int32)
    pad_dst = N + (jnp.arange(SEG_PAD - SEG, dtype=jnp.int32) % (N_PAD - N))
    pad_blk = jnp.broadcast_to(
        jnp.stack([pad_src, pad_dst])[:, None, :], (2, 32, SEG_PAD - SEG))
    ep_pad = jnp.concatenate([ep, pad_blk], axis=2).reshape(2, E_PAD)

    x_pad = jnp.pad(features, ((0, N_PAD - N), (0, 0)))
    ones1 = jnp.ones((K,), jnp.float32)
    zeros1 = jnp.zeros((N_PAD,), jnp.float32)
    zeros_t = jnp.zeros((N_PAD, D_OUT), jnp.float32)

    deg_f = _sc_degrees(ep_pad, ones1, zeros1)
    ps = deg_f[:N_PAD].reshape(N_PAD, 1)
    pd = deg_f[N_PAD:].reshape(N_PAD, 1)

    h1, ns, nd = _tc1(x_pad, W1, ps, pd)

    agg1_f = _edge_agg_l1(h1.reshape(NC * N_PAD, D_OUT), ep_pad, zeros_t)
    agg1 = agg1_f.reshape(NC, N_PAD, D_OUT)

    h2 = _tc2(agg1, ns, nd, b1.reshape(1, NC, D_OUT),
              W2.reshape(NC, D_OUT, D_OUT))

    agg2_f = _edge_agg_l2(h2, ep_pad, zeros_t)
    agg2 = agg2_f.reshape(NC, N_PAD, D_OUT)

    logits = _tc3(agg2, nd, b2.reshape(1, D_OUT))
    return logits[:N]
